# Initial kernel scaffold; baseline (speedup 1.0000x reference)
import jax
import jax.numpy as jnp
from jax.experimental import pallas as pl


def kernel(x, edge_index, batch, W1, att_src1, att_dst1, b1, W2, att_src2, att_dst2, b2):
    raise NotImplementedError("write your pallas kernel here")


# speedup vs baseline: 14.8776x; 14.8776x over previous
import functools

import jax
import jax.numpy as jnp
from jax import lax
from jax.experimental import pallas as pl
from jax.experimental.pallas import tpu as pltpu
from jax.experimental.pallas import tpu_sc as plsc

N = 10000
E = 160000
D_IN = 256
HID = 128
HEADS = 4
D_OUT = 256
G = 64

NC = 2
NS = 16
LANES = 16
NW = NC * NS
EB = 64
PB = 128
NB_E = E // EB
N_PAD = 10240
NPT = N_PAD // NS
DEN_R = N_PAD // 128
BM = 1000


def _tc1_body(x_ref, w_ref, as_ref, ad_ref, t0, t1, t2, t3, aso, ado):
    xw = jnp.dot(x_ref[...], w_ref[...], preferred_element_type=jnp.float32)
    t0[...] = xw[:, 0:128]
    t1[...] = xw[:, 128:256]
    t2[...] = xw[:, 256:384]
    t3[...] = xw[:, 384:512]
    aso[...] = jnp.dot(xw, as_ref[...], preferred_element_type=jnp.float32)
    ado[...] = jnp.dot(xw, ad_ref[...], preferred_element_type=jnp.float32)


def _tc2_body(p0, p1, p2, p3, den_ref, b1_ref, w2_ref, as2_ref, ad2_ref,
              t20, t21, as2o, ad2o):
    d = den_ref[0] + den_ref[1] + 1e-16
    hs = []
    for h, p in enumerate((p0, p1, p2, p3)):
        hs.append((p[0] + p[1]) / d[:, h:h + 1])
    hcat = jnp.concatenate(hs, axis=1) + b1_ref[...]
    hcat = jnp.where(hcat > 0, hcat, jnp.exp(jnp.minimum(hcat, 0.0)) - 1.0)
    t2 = jnp.dot(hcat, w2_ref[...], preferred_element_type=jnp.float32)
    t20[...] = t2[:, 0:128]
    t21[...] = t2[:, 128:256]
    as2o[...] = jnp.dot(t2, as2_ref[...], preferred_element_type=jnp.float32)
    ad2o[...] = jnp.dot(t2, ad2_ref[...], preferred_element_type=jnp.float32)


def _tc3_body(q0, q1, den2_ref, b2_ref, h0_out, h1_out):
    d = den2_ref[0] + den2_ref[1] + 1e-16
    b2v = b2_ref[...]
    h0_out[...] = (q0[0] + q0[1]) / d + b2v[:, 0:128]
    h1_out[...] = (q1[0] + q1[1]) / d + b2v[:, 128:256]


def _tc4_body(p0_ref, p1_ref, c_ref, out_ref):
    cnt = jnp.maximum((c_ref[0] + c_ref[1])[:, 0:1], 1.0)
    out_ref[...] = jnp.concatenate(
        [p0_ref[0] + p0_ref[1], p1_ref[0] + p1_ref[1]], axis=1) / cnt


def _sc_mesh():
    return plsc.VectorSubcoreMesh(core_axis_name="c", subcore_axis_name="s",
                                  num_cores=NC, num_subcores=NS)


def _make_edge_kernel():
    @functools.partial(
        pl.kernel,
        out_type=(jax.ShapeDtypeStruct((NC, N_PAD, HID), jnp.float32),
                  jax.ShapeDtypeStruct((NC, DEN_R, 128), jnp.float32)),
        mesh=_sc_mesh(),
        compiler_params=pltpu.CompilerParams(needs_layout_passes=False),
        scratch_types=[
            pltpu.VMEM((N,), jnp.float32),
            pltpu.VMEM((N,), jnp.float32),
            pltpu.VMEM((DEN_R, 128), jnp.float32),
            pltpu.VMEM((DEN_R,), jnp.int32),
            pltpu.VMEM((EB,), jnp.int32),
            pltpu.VMEM((EB,), jnp.int32),
            pltpu.VMEM((EB,), jnp.float32),
            pltpu.VMEM((EB, HID), jnp.float32),
            pltpu.VMEM_SHARED((N_PAD, HID), jnp.float32),
            pltpu.VMEM_SHARED((DEN_R, 128), jnp.float32),
            pltpu.SemaphoreType.DMA,
        ],
    )
    def edge_kernel(asrc_hbm, adst_hbm, src_hbm, dst_hbm, table_hbm, rowid_hbm,
                    out_num, out_den,
                    asrc_v, adst_v, den_v, rid_v, src_v, dst_v, w_v, rows_v,
                    num_sp, den_sp, sem):
        c = lax.axis_index("c")
        s = lax.axis_index("s")
        wid = c * NS + s

        pltpu.sync_copy(asrc_hbm, asrc_v)
        pltpu.sync_copy(adst_hbm, adst_v)
        pltpu.sync_copy(rowid_hbm, rid_v)

        zero16 = jnp.zeros((LANES,), jnp.float32)

        def _zrow(r, _):
            for cc in range(HID // LANES):
                rows_v[r, pl.ds(cc * LANES, LANES)] = zero16
            return 0
        lax.fori_loop(0, EB, _zrow, 0)

        def _zden(r, _):
            for cc in range(128 // LANES):
                den_v[r, pl.ds(cc * LANES, LANES)] = zero16
            return 0
        lax.fori_loop(0, DEN_R, _zden, 0)

        nbase = s * NPT
        for k in range(NPT // EB):
            pltpu.sync_copy(rows_v, num_sp.at[pl.ds(nbase + k * EB, EB)])

        @pl.when(s < DEN_R // 8)
        def _():
            pltpu.sync_copy(den_v.at[pl.ds(0, 8)], den_sp.at[pl.ds(s * 8, 8)])
        plsc.subcore_barrier()

        nfull = NB_E // NW
        nb = jnp.where(wid < NB_E - nfull * NW, nfull + 1, nfull)

        def batch_body(j, _):
            base = (wid + j * NW) * EB
            pltpu.sync_copy(src_hbm.at[pl.ds(base, EB)], src_v)
            pltpu.sync_copy(dst_hbm.at[pl.ds(base, EB)], dst_v)
            cp = pltpu.async_copy(table_hbm.at[src_v], rows_v, sem)
            for t in range(EB // LANES):
                s16 = src_v[pl.ds(t * LANES, LANES)]
                d16 = dst_v[pl.ds(t * LANES, LANES)]
                e = plsc.load_gather(asrc_v, [s16]) + plsc.load_gather(adst_v, [d16])
                w = jnp.exp(jnp.maximum(e, 0.2 * e))
                w_v[pl.ds(t * LANES, LANES)] = w
                plsc.addupdate_scatter(
                    den_v,
                    [lax.shift_right_logical(d16, 7), lax.bitwise_and(d16, 127)],
                    w)
            cp.wait()

            def row_body(r, _):
                wb = plsc.load_gather(w_v, [jnp.zeros((LANES,), jnp.int32) + r])
                for cc in range(HID // LANES):
                    sl = pl.ds(cc * LANES, LANES)
                    rows_v[r, sl] = rows_v[r, sl] * wb
                return 0
            lax.fori_loop(0, EB, row_body, 0)
            pltpu.sync_copy(rows_v, num_sp.at[dst_v], add=True)
            return 0
        lax.fori_loop(0, nb, batch_body, 0)

        pltpu.sync_copy(den_v, den_sp.at[rid_v], add=True)
        plsc.subcore_barrier()

        for k in range(NPT // EB):
            pltpu.sync_copy(num_sp.at[pl.ds(nbase + k * EB, EB)],
                            out_num.at[c, pl.ds(nbase + k * EB, EB)])

        @pl.when(s < DEN_R // 8)
        def _():
            pltpu.sync_copy(den_sp.at[pl.ds(s * 8, 8)],
                            out_den.at[c, pl.ds(s * 8, 8)])

    return edge_kernel


def _make_pool_kernel():
    NBP = N // PB
    TAIL = N - NBP * PB

    @functools.partial(
        pl.kernel,
        out_type=(jax.ShapeDtypeStruct((NC, G, 128), jnp.float32),
                  jax.ShapeDtypeStruct((NC, G, 128), jnp.float32),
                  jax.ShapeDtypeStruct((NC, G, 128), jnp.float32)),
        mesh=_sc_mesh(),
        compiler_params=pltpu.CompilerParams(needs_layout_passes=False),
        scratch_types=[
            pltpu.VMEM((PB, 128), jnp.float32),
            pltpu.VMEM((PB, 128), jnp.float32),
            pltpu.VMEM((PB,), jnp.int32),
            pltpu.VMEM((TAIL, 128), jnp.float32),
            pltpu.VMEM((TAIL, 128), jnp.float32),
            pltpu.VMEM((TAIL,), jnp.int32),
            pltpu.VMEM((G, 128), jnp.float32),
            pltpu.VMEM((G,), jnp.int32),
            pltpu.VMEM_SHARED((G, 128), jnp.float32),
            pltpu.VMEM_SHARED((G, 128), jnp.float32),
            pltpu.VMEM_SHARED((G, 128), jnp.float32),
        ],
    )
    def pool_kernel(h0_hbm, h1_hbm, batch_hbm, gid_hbm,
                    out_p0, out_p1, out_cnt,
                    rows0_v, rows1_v, bid_v, trows0_v, trows1_v, tbid_v,
                    cnt_v, rid_v, p0_sp, p1_sp, cnt_sp):
        c = lax.axis_index("c")
        s = lax.axis_index("s")
        wid = c * NS + s

        pltpu.sync_copy(gid_hbm, rid_v)
        zero16 = jnp.zeros((LANES,), jnp.float32)
        one16 = jnp.ones((LANES,), jnp.float32)

        def _zcnt(r, _):
            for cc in range(128 // LANES):
                cnt_v[r, pl.ds(cc * LANES, LANES)] = zero16
                rows0_v[r, pl.ds(cc * LANES, LANES)] = zero16
            return 0
        lax.fori_loop(0, G, _zcnt, 0)

        @pl.when(s < G // 8)
        def _():
            pltpu.sync_copy(rows0_v.at[pl.ds(0, 8)], p0_sp.at[pl.ds(s * 8, 8)])
            pltpu.sync_copy(rows0_v.at[pl.ds(0, 8)], p1_sp.at[pl.ds(s * 8, 8)])
            pltpu.sync_copy(cnt_v.at[pl.ds(0, 8)], cnt_sp.at[pl.ds(s * 8, 8)])
        plsc.subcore_barrier()

        nfull = NBP // NW
        nb = jnp.where(wid < NBP - nfull * NW, nfull + 1, nfull)

        def pb(j, _):
            base = (wid + j * NW) * PB
            pltpu.sync_copy(h0_hbm.at[pl.ds(base, PB)], rows0_v)
            pltpu.sync_copy(h1_hbm.at[pl.ds(base, PB)], rows1_v)
            pltpu.sync_copy(batch_hbm.at[pl.ds(base, PB)], bid_v)
            for t in range(PB // LANES):
                b16 = bid_v[pl.ds(t * LANES, LANES)]
                plsc.addupdate_scatter(
                    cnt_v, [b16, jnp.zeros((LANES,), jnp.int32)], one16)
            pltpu.sync_copy(rows0_v, p0_sp.at[bid_v], add=True)
            pltpu.sync_copy(rows1_v, p1_sp.at[bid_v], add=True)
            return 0
        lax.fori_loop(0, nb, pb, 0)

        @pl.when(wid == NW - 1)
        def _():
            pltpu.sync_copy(h0_hbm.at[pl.ds(N - TAIL, TAIL)], trows0_v)
            pltpu.sync_copy(h1_hbm.at[pl.ds(N - TAIL, TAIL)], trows1_v)
            pltpu.sync_copy(batch_hbm.at[pl.ds(N - TAIL, TAIL)], tbid_v)
            t16 = tbid_v[pl.ds(0, LANES)]
            plsc.addupdate_scatter(
                cnt_v, [t16, jnp.zeros((LANES,), jnp.int32)], one16)
            pltpu.sync_copy(trows0_v, p0_sp.at[tbid_v], add=True)
            pltpu.sync_copy(trows1_v, p1_sp.at[tbid_v], add=True)

        pltpu.sync_copy(cnt_v, cnt_sp.at[rid_v], add=True)
        plsc.subcore_barrier()

        @pl.when(s < G // 8)
        def _():
            pltpu.sync_copy(p0_sp.at[pl.ds(s * 8, 8)],
                            out_p0.at[c, pl.ds(s * 8, 8)])
            pltpu.sync_copy(p1_sp.at[pl.ds(s * 8, 8)],
                            out_p1.at[c, pl.ds(s * 8, 8)])
            pltpu.sync_copy(cnt_sp.at[pl.ds(s * 8, 8)],
                            out_cnt.at[c, pl.ds(s * 8, 8)])

    return pool_kernel


def kernel(x, edge_index, batch, W1, att_src1, att_dst1, b1,
           W2, att_src2, att_dst2, b2):
    x = x.astype(jnp.float32)
    src = edge_index[0]
    dst = edge_index[1]
    rowids = jnp.arange(DEN_R, dtype=jnp.int32)

    eyeH = jnp.eye(HEADS, dtype=jnp.float32)
    As1 = jnp.einsum("hd,hg->hdg", att_src1, eyeH).reshape(HEADS * HID, HEADS)
    Ad1 = jnp.einsum("hd,hg->hdg", att_dst1, eyeH).reshape(HEADS * HID, HEADS)
    As2 = att_src2.reshape(D_OUT, 1)
    Ad2 = att_dst2.reshape(D_OUT, 1)

    grid = (N // BM,)
    row_spec = lambda w: pl.BlockSpec((BM, w), lambda i: (i, 0))
    full_spec = lambda a, b_: pl.BlockSpec((a, b_), lambda i: (0, 0))
    t10, t11, t12, t13, as1, ad1 = pl.pallas_call(
        _tc1_body,
        grid=grid,
        in_specs=[row_spec(D_IN), full_spec(D_IN, HEADS * HID),
                  full_spec(HEADS * HID, HEADS), full_spec(HEADS * HID, HEADS)],
        out_specs=[row_spec(HID)] * 4 + [row_spec(HEADS)] * 2,
        out_shape=[jax.ShapeDtypeStruct((N, HID), jnp.float32)] * 4
                  + [jax.ShapeDtypeStruct((N, HEADS), jnp.float32)] * 2,
    )(x, W1, As1, Ad1)

    edge_k = _make_edge_kernel()
    tables1 = (t10, t11, t12, t13)
    nums1, dens1 = [], []
    for h in range(HEADS):
        on, od = edge_k(as1[:, h], ad1[:, h], src, dst, tables1[h], rowids)
        nums1.append(on)
        dens1.append(od)
    den1 = jnp.stack([p.reshape(NC, N_PAD) for p in dens1], axis=-1)

    part_spec = pl.BlockSpec((NC, BM, HID), lambda i: (0, i, 0))
    t20, t21, as2, ad2 = pl.pallas_call(
        _tc2_body,
        grid=grid,
        in_specs=[part_spec] * 4
                 + [pl.BlockSpec((NC, BM, HEADS), lambda i: (0, i, 0)),
                    full_spec(1, HEADS * HID),
                    full_spec(HEADS * HID, D_OUT),
                    full_spec(D_OUT, 1), full_spec(D_OUT, 1)],
        out_specs=[row_spec(HID)] * 2 + [row_spec(1)] * 2,
        out_shape=[jax.ShapeDtypeStruct((N, HID), jnp.float32)] * 2
                  + [jax.ShapeDtypeStruct((N, 1), jnp.float32)] * 2,
    )(nums1[0], nums1[1], nums1[2], nums1[3], den1, b1.reshape(1, HEADS * HID),
      W2, As2, Ad2)

    q0n, q0d = edge_k(as2[:, 0], ad2[:, 0], src, dst, t20, rowids)
    q1n, _ = edge_k(as2[:, 0], ad2[:, 0], src, dst, t21, rowids)
    den2 = q0d.reshape(NC, N_PAD)[:, :, None]

    h0, h1 = pl.pallas_call(
        _tc3_body,
        grid=grid,
        in_specs=[part_spec, part_spec,
                  pl.BlockSpec((NC, BM, 1), lambda i: (0, i, 0)),
                  full_spec(1, D_OUT)],
        out_specs=[row_spec(HID)] * 2,
        out_shape=[jax.ShapeDtypeStruct((N, HID), jnp.float32)] * 2,
    )(q0n, q1n, den2, b2.reshape(1, D_OUT))

    pool_k = _make_pool_kernel()
    gids = jnp.arange(G, dtype=jnp.int32)
    pool_p0, pool_p1, cnt_part = pool_k(h0, h1, batch, gids)

    pooled = pl.pallas_call(
        _tc4_body,
        grid=(1,),
        in_specs=[pl.BlockSpec((NC, G, 128), lambda i: (0, 0, 0))] * 3,
        out_specs=pl.BlockSpec((G, D_OUT), lambda i: (0, 0)),
        out_shape=jax.ShapeDtypeStruct((G, D_OUT), jnp.float32),
    )(pool_p0, pool_p1, cnt_part)
    return pooled
